# Initial kernel scaffold; baseline (speedup 1.0000x reference)
import jax
import jax.numpy as jnp
from jax.experimental import pallas as pl


def kernel(x, table):
    raise NotImplementedError("write your pallas kernel here")


# speedup vs baseline: 1.0424x; 1.0424x over previous
import jax
import jax.numpy as jnp
from jax.experimental import pallas as pl
from jax.experimental.pallas import tpu as pltpu
from jax.experimental.pallas import tpu_sc as plsc

EMBED_DIM = 32
WINDOW = 128


def kernel(x, table):
    batch, seq = x.shape
    num_idx = batch * seq
    idx = x.reshape(1, num_idx)

    mesh = plsc.VectorSubcoreMesh(core_axis_name="core", subcore_axis_name="subcore")

    @pl.kernel(
        out_type=jax.ShapeDtypeStruct((num_idx, EMBED_DIM), table.dtype),
        mesh=mesh,
        compiler_params=pltpu.CompilerParams(use_tc_tiling_on_sc=False),
    )
    def gather_kernel(table_hbm, idx_hbm, out_hbm):
        def body(idx_vmem, out_vmem):
            pltpu.sync_copy(table_hbm.at[idx_vmem.at[0]], out_vmem)

        pltpu.emit_pipeline(
            body,
            grid=(num_idx // WINDOW,),
            in_specs=[pl.BlockSpec((1, WINDOW), index_map=lambda i: (0, i))],
            out_specs=[pl.BlockSpec((WINDOW, EMBED_DIM), index_map=lambda i: (i, 0))],
            core_axis_name=("core", "subcore"),
            dimension_semantics=(pltpu.PARALLEL,),
        )(idx_hbm, out_hbm)

    out = gather_kernel(table, idx)
    return out.reshape(batch, seq, EMBED_DIM)


# speedup vs baseline: 1.1117x; 1.0665x over previous
import jax
import jax.numpy as jnp
from jax import lax
from jax.experimental import pallas as pl
from jax.experimental.pallas import tpu as pltpu
from jax.experimental.pallas import tpu_sc as plsc

EMBED_DIM = 32
CHUNK = 128
NBUF = 8
NUM_WORKERS = 32


def kernel(x, table):
    batch, seq = x.shape
    num_idx = batch * seq
    chunks_per_worker = num_idx // (NUM_WORKERS * CHUNK)
    rounds = chunks_per_worker // NBUF
    idx = x.reshape(NUM_WORKERS, chunks_per_worker, CHUNK)

    mesh = plsc.VectorSubcoreMesh(core_axis_name="core", subcore_axis_name="subcore")

    scratch = (
        [pltpu.VMEM((chunks_per_worker, CHUNK), jnp.int32)]
        + [pltpu.VMEM((CHUNK, EMBED_DIM), jnp.float32) for _ in range(NBUF)]
        + [pltpu.SemaphoreType.DMA for _ in range(2 * NBUF)]
    )

    @pl.kernel(
        out_type=jax.ShapeDtypeStruct((num_idx, EMBED_DIM), table.dtype),
        mesh=mesh,
        scratch_types=scratch,
        compiler_params=pltpu.CompilerParams(use_tc_tiling_on_sc=False),
    )
    def gather_kernel(table_hbm, idx_hbm, out_hbm, idx_v, *rest):
        bufs = rest[:NBUF]
        gsem = rest[NBUF:2 * NBUF]
        wsem = rest[2 * NBUF:]

        wid = lax.axis_index("subcore") * 2 + lax.axis_index("core")
        row0 = wid * (chunks_per_worker * CHUNK)

        pltpu.sync_copy(idx_hbm.at[wid], idx_v)

        for b in range(NBUF):
            pltpu.async_copy(table_hbm.at[idx_v.at[b]], bufs[b], gsem[b])

        @pl.loop(0, rounds - 1)
        def _(g):
            base = g * NBUF
            for b in range(NBUF):
                j = base + b
                pltpu.make_async_copy(table_hbm.at[idx_v.at[j]], bufs[b], gsem[b]).wait()
                pltpu.async_copy(
                    bufs[b], out_hbm.at[pl.ds(row0 + j * CHUNK, CHUNK), :], wsem[b]
                )
            for b in range(NBUF):
                j = base + NBUF + b
                pltpu.make_async_copy(
                    bufs[b], out_hbm.at[pl.ds(row0 + (j - NBUF) * CHUNK, CHUNK), :], wsem[b]
                ).wait()
                pltpu.async_copy(table_hbm.at[idx_v.at[j]], bufs[b], gsem[b])

        base = (rounds - 1) * NBUF
        for b in range(NBUF):
            j = base + b
            pltpu.make_async_copy(table_hbm.at[idx_v.at[j]], bufs[b], gsem[b]).wait()
            pltpu.async_copy(
                bufs[b], out_hbm.at[pl.ds(row0 + j * CHUNK, CHUNK), :], wsem[b]
            )
        for b in range(NBUF):
            j = base + b
            pltpu.make_async_copy(
                bufs[b], out_hbm.at[pl.ds(row0 + j * CHUNK, CHUNK), :], wsem[b]
            ).wait()

    out = gather_kernel(table, idx)
    return out.reshape(batch, seq, EMBED_DIM)


# speedup vs baseline: 1.7736x; 1.5954x over previous
import jax
import jax.numpy as jnp
from jax import lax
from jax.experimental import pallas as pl
from jax.experimental.pallas import tpu as pltpu
from jax.experimental.pallas import tpu_sc as plsc

EMBED_DIM = 32
NBUF = 8
NUM_WORKERS = 32


def kernel(x, table):
    batch, seq = x.shape
    rows_per_worker = batch // NUM_WORKERS
    rounds = rows_per_worker // NBUF

    mesh = plsc.VectorSubcoreMesh(core_axis_name="core", subcore_axis_name="subcore")

    scratch = (
        [pltpu.VMEM((rows_per_worker, seq), jnp.int32)]
        + [pltpu.VMEM((seq, EMBED_DIM), jnp.float32) for _ in range(NBUF)]
        + [pltpu.SemaphoreType.DMA for _ in range(2 * NBUF + 1)]
    )

    @pl.kernel(
        out_type=jax.ShapeDtypeStruct((batch, seq, EMBED_DIM), table.dtype),
        mesh=mesh,
        scratch_types=scratch,
        compiler_params=pltpu.CompilerParams(use_tc_tiling_on_sc=False),
    )
    def gather_kernel(table_hbm, idx_hbm, out_hbm, idx_v, *rest):
        bufs = rest[:NBUF]
        gsem = rest[NBUF:2 * NBUF]
        wsem = rest[2 * NBUF:3 * NBUF]
        isem = rest[3 * NBUF]

        wid = lax.axis_index("subcore") * 2 + lax.axis_index("core")
        b0 = wid * rows_per_worker

        pltpu.async_copy(idx_hbm.at[pl.ds(b0, rows_per_worker), :], idx_v, isem).wait()

        def start_gather(j, b):
            pltpu.async_copy(table_hbm.at[idx_v.at[j]], bufs[b], gsem[b])

        def wait_gather(j, b):
            pltpu.make_async_copy(
                table_hbm.at[idx_v.at[j]], bufs[b], gsem[b]
            ).wait()

        def start_write(j, b):
            pltpu.async_copy(bufs[b], out_hbm.at[b0 + j], wsem[b])

        def wait_write(j, b):
            pltpu.make_async_copy(bufs[b], out_hbm.at[b0 + j], wsem[b]).wait()

        for b in range(NBUF):
            start_gather(b, b)

        @pl.loop(0, rounds - 1)
        def _(g):
            base = g * NBUF
            for b in range(NBUF):
                wait_gather(base + b, b)
                start_write(base + b, b)
            for b in range(NBUF):
                wait_write(base + b, b)
                start_gather(base + NBUF + b, b)

        base = (rounds - 1) * NBUF
        for b in range(NBUF):
            wait_gather(base + b, b)
            start_write(base + b, b)
        for b in range(NBUF):
            wait_write(base + b, b)

    return gather_kernel(table, x)
